# baseline (speedup 1.0000x reference)
import functools

import jax
import jax.numpy as jnp
from jax import lax
from jax.experimental import pallas as pl
from jax.experimental.pallas import tpu as pltpu
from jax.experimental.pallas import tpu_sc as plsc

N_ENV = 1024
BUF = 512
N_OBS = 64
N_ACT = 16
BATCH = 256

NC = 2
NS = 16
NW = NC * NS
B = N_ENV * BATCH
RPW = B // NW
EPW = N_ENV // NW
SLAB = EPW * BUF
CHUNK = 128
NPAIR = RPW // (2 * CHUNK)


def _gather_start(table, idx_v, c, buf, sem):
    ids = idx_v.at[pl.ds(c * CHUNK, CHUNK)]
    pltpu.make_async_copy(table.at[ids], buf, sem).start()


def _gather_wait(table, idx_v, c, buf, sem):
    ids = idx_v.at[pl.ds(c * CHUNK, CHUNK)]
    pltpu.make_async_copy(table.at[ids], buf, sem).wait()


@functools.partial(
    pl.kernel,
    mesh=plsc.VectorSubcoreMesh(core_axis_name="c", subcore_axis_name="s"),
    compiler_params=pltpu.CompilerParams(
        needs_layout_passes=False, use_tc_tiling_on_sc=False),
    out_type=[
        jax.ShapeDtypeStruct((B, N_OBS), jnp.float32),
        jax.ShapeDtypeStruct((B, N_ACT), jnp.float32),
        jax.ShapeDtypeStruct((B,), jnp.float32),
        jax.ShapeDtypeStruct((B,), jnp.int32),
        jax.ShapeDtypeStruct((B,), jnp.int32),
        jax.ShapeDtypeStruct((B, N_OBS), jnp.float32),
    ],
    scratch_types=[
        pltpu.VMEM((RPW,), jnp.int32),
        pltpu.VMEM((SLAB,), jnp.float32),
        pltpu.VMEM((SLAB,), jnp.int32),
        pltpu.VMEM((SLAB,), jnp.int32),
        pltpu.VMEM((RPW,), jnp.float32),
        pltpu.VMEM((RPW,), jnp.int32),
        pltpu.VMEM((RPW,), jnp.int32),
        pltpu.VMEM((CHUNK, N_OBS), jnp.float32),
        pltpu.VMEM((CHUNK, N_OBS), jnp.float32),
        pltpu.VMEM((CHUNK, N_OBS), jnp.float32),
        pltpu.VMEM((CHUNK, N_OBS), jnp.float32),
        pltpu.VMEM((CHUNK, N_ACT), jnp.float32),
        pltpu.VMEM((CHUNK, N_ACT), jnp.float32),
        pltpu.SemaphoreType.DMA,
        pltpu.SemaphoreType.DMA,
        pltpu.SemaphoreType.DMA,
        pltpu.SemaphoreType.DMA,
    ],
)
def _sample(obs, nobs, act, rew, don, trn, idx,
            obs_o, act_o, rew_o, don_o, trn_o, nobs_o,
            idx_v, rew_v, don_v, trn_v, rew_ov, don_ov, trn_ov,
            obs_b0, obs_b1, nob_b0, nob_b1, act_b0, act_b1,
            sem_i, sem_t, sem0, sem1):
    wid = lax.axis_index("s") * NC + lax.axis_index("c")
    row0 = wid * RPW
    tab0 = wid * SLAB

    pltpu.make_async_copy(idx.at[pl.ds(row0, RPW)], idx_v, sem_i).start()
    pltpu.make_async_copy(rew.at[pl.ds(tab0, SLAB)], rew_v, sem_t).start()
    pltpu.make_async_copy(don.at[pl.ds(tab0, SLAB)], don_v, sem_t).start()
    pltpu.make_async_copy(trn.at[pl.ds(tab0, SLAB)], trn_v, sem_t).start()
    pltpu.make_async_copy(idx.at[pl.ds(row0, RPW)], idx_v, sem_i).wait()

    env0 = wid * EPW
    vpe = BATCH // 16

    def mk_gidx(j, _):
        off = (env0 + j // vpe) * BUF
        s = pl.ds(j * 16, 16)
        idx_v[s] = idx_v[s] + off
        return 0

    lax.fori_loop(0, RPW // 16, mk_gidx, 0)

    _gather_start(obs, idx_v, 0, obs_b0, sem0)
    _gather_start(nobs, idx_v, 0, nob_b0, sem0)
    _gather_start(act, idx_v, 0, act_b0, sem0)

    def pair(g, _):
        c0 = 2 * g
        c1 = 2 * g + 1
        _gather_start(obs, idx_v, c1, obs_b1, sem1)
        _gather_start(nobs, idx_v, c1, nob_b1, sem1)
        _gather_start(act, idx_v, c1, act_b1, sem1)
        _gather_wait(obs, idx_v, c0, obs_b0, sem0)
        _gather_wait(nobs, idx_v, c0, nob_b0, sem0)
        _gather_wait(act, idx_v, c0, act_b0, sem0)
        o0 = row0 + c0 * CHUNK
        pltpu.sync_copy(obs_b0, obs_o.at[pl.ds(o0, CHUNK)])
        pltpu.sync_copy(nob_b0, nobs_o.at[pl.ds(o0, CHUNK)])
        pltpu.sync_copy(act_b0, act_o.at[pl.ds(o0, CHUNK)])

        @pl.when(g < NPAIR - 1)
        def _():
            _gather_start(obs, idx_v, c1 + 1, obs_b0, sem0)
            _gather_start(nobs, idx_v, c1 + 1, nob_b0, sem0)
            _gather_start(act, idx_v, c1 + 1, act_b0, sem0)

        _gather_wait(obs, idx_v, c1, obs_b1, sem1)
        _gather_wait(nobs, idx_v, c1, nob_b1, sem1)
        _gather_wait(act, idx_v, c1, act_b1, sem1)
        o1 = row0 + c1 * CHUNK
        pltpu.sync_copy(obs_b1, obs_o.at[pl.ds(o1, CHUNK)])
        pltpu.sync_copy(nob_b1, nobs_o.at[pl.ds(o1, CHUNK)])
        pltpu.sync_copy(act_b1, act_o.at[pl.ds(o1, CHUNK)])
        return 0

    lax.fori_loop(0, NPAIR, pair, 0)

    pltpu.make_async_copy(rew.at[pl.ds(tab0, SLAB)], rew_v, sem_t).wait()
    pltpu.make_async_copy(don.at[pl.ds(tab0, SLAB)], don_v, sem_t).wait()
    pltpu.make_async_copy(trn.at[pl.ds(tab0, SLAB)], trn_v, sem_t).wait()

    def sgather(j, _):
        s = pl.ds(j * 16, 16)
        li = idx_v[s] - tab0
        rew_ov[s] = plsc.load_gather(rew_v, [li])
        don_ov[s] = plsc.load_gather(don_v, [li])
        trn_ov[s] = plsc.load_gather(trn_v, [li])
        return 0

    lax.fori_loop(0, RPW // 16, sgather, 0)

    pltpu.sync_copy(rew_ov, rew_o.at[pl.ds(row0, RPW)])
    pltpu.sync_copy(don_ov, don_o.at[pl.ds(row0, RPW)])
    pltpu.sync_copy(trn_ov, trn_o.at[pl.ds(row0, RPW)])


def kernel(observations, actions, rewards, dones, truncations,
           next_observations, indices):
    obs = observations.reshape(N_ENV * BUF, N_OBS)
    nobs = next_observations.reshape(N_ENV * BUF, N_OBS)
    act = actions.reshape(N_ENV * BUF, N_ACT)
    rew = rewards.reshape(-1)
    don = dones.reshape(-1)
    trn = truncations.reshape(-1)
    idx = indices.reshape(-1)
    obs_s, act_s, rew_s, don_s, trn_s, nobs_s = _sample(
        obs, nobs, act, rew, don, trn, idx)
    eff = jnp.ones((B,), jnp.int32)
    return (obs_s, act_s, rew_s, don_s, trn_s, nobs_s, eff)


# speedup vs baseline: 1.0059x; 1.0059x over previous
import functools

import jax
import jax.numpy as jnp
from jax import lax
from jax.experimental import pallas as pl
from jax.experimental.pallas import tpu as pltpu
from jax.experimental.pallas import tpu_sc as plsc

N_ENV = 1024
BUF = 512
N_OBS = 64
N_ACT = 16
BATCH = 256

NC = 2
NS = 16
NW = NC * NS
B = N_ENV * BATCH
RPW = B // NW
EPW = N_ENV // NW
SLAB = EPW * BUF
CHUNK = 128
NPAIR = RPW // (2 * CHUNK)


def _gather_start(table, idx_v, c, buf, sem):
    ids = idx_v.at[pl.ds(c * CHUNK, CHUNK)]
    pltpu.make_async_copy(table.at[ids], buf, sem).start()


def _gather_wait(table, idx_v, c, buf, sem):
    ids = idx_v.at[pl.ds(c * CHUNK, CHUNK)]
    pltpu.make_async_copy(table.at[ids], buf, sem).wait()


@functools.partial(
    pl.kernel,
    mesh=plsc.VectorSubcoreMesh(core_axis_name="c", subcore_axis_name="s"),
    compiler_params=pltpu.CompilerParams(
        needs_layout_passes=False, use_tc_tiling_on_sc=False),
    out_type=[
        jax.ShapeDtypeStruct((B, N_OBS), jnp.float32),
        jax.ShapeDtypeStruct((B, N_ACT), jnp.float32),
        jax.ShapeDtypeStruct((B,), jnp.float32),
        jax.ShapeDtypeStruct((B,), jnp.int32),
        jax.ShapeDtypeStruct((B,), jnp.int32),
        jax.ShapeDtypeStruct((B, N_OBS), jnp.float32),
    ],
    scratch_types=[
        pltpu.VMEM((RPW,), jnp.int32),
        pltpu.VMEM((SLAB,), jnp.float32),
        pltpu.VMEM((SLAB,), jnp.int32),
        pltpu.VMEM((SLAB,), jnp.int32),
        pltpu.VMEM((RPW,), jnp.float32),
        pltpu.VMEM((RPW,), jnp.int32),
        pltpu.VMEM((RPW,), jnp.int32),
        pltpu.VMEM((CHUNK, N_OBS), jnp.float32),
        pltpu.VMEM((CHUNK, N_OBS), jnp.float32),
        pltpu.VMEM((CHUNK, N_OBS), jnp.float32),
        pltpu.VMEM((CHUNK, N_OBS), jnp.float32),
        pltpu.VMEM((CHUNK, N_ACT), jnp.float32),
        pltpu.VMEM((CHUNK, N_ACT), jnp.float32),
        pltpu.SemaphoreType.DMA,
        pltpu.SemaphoreType.DMA,
        pltpu.SemaphoreType.DMA,
        pltpu.SemaphoreType.DMA,
    ],
)
def _sample(obs, nobs, act, rew, don, trn, idx,
            obs_o, act_o, rew_o, don_o, trn_o, nobs_o,
            idx_v, rew_v, don_v, trn_v, rew_ov, don_ov, trn_ov,
            obs_b0, obs_b1, nob_b0, nob_b1, act_b0, act_b1,
            sem_i, sem_t, sem0, sem1):
    wid = lax.axis_index("s") * NC + lax.axis_index("c")
    row0 = wid * RPW
    tab0 = wid * SLAB

    pltpu.make_async_copy(idx.at[pl.ds(row0, RPW)], idx_v, sem_i).start()
    pltpu.make_async_copy(rew.at[pl.ds(tab0, SLAB)], rew_v, sem_t).start()
    pltpu.make_async_copy(don.at[pl.ds(tab0, SLAB)], don_v, sem_t).start()
    pltpu.make_async_copy(trn.at[pl.ds(tab0, SLAB)], trn_v, sem_t).start()
    pltpu.make_async_copy(idx.at[pl.ds(row0, RPW)], idx_v, sem_i).wait()

    env0 = wid * EPW
    vpe = BATCH // 16

    def mk_gidx(j, _):
        off = (env0 + j // vpe) * BUF
        s = pl.ds(j * 16, 16)
        idx_v[s] = idx_v[s] + off
        return 0

    lax.fori_loop(0, RPW // 16, mk_gidx, 0)

    pltpu.make_async_copy(rew.at[pl.ds(tab0, SLAB)], rew_v, sem_t).wait()
    pltpu.make_async_copy(don.at[pl.ds(tab0, SLAB)], don_v, sem_t).wait()
    pltpu.make_async_copy(trn.at[pl.ds(tab0, SLAB)], trn_v, sem_t).wait()

    def sgather(j, _):
        s = pl.ds(j * 16, 16)
        li = idx_v[s] - tab0
        rew_ov[s] = plsc.load_gather(rew_v, [li])
        don_ov[s] = plsc.load_gather(don_v, [li])
        trn_ov[s] = plsc.load_gather(trn_v, [li])
        return 0

    SG_PER_PAIR = (RPW // 16) // NPAIR

    _gather_start(obs, idx_v, 0, obs_b0, sem0)
    _gather_start(nobs, idx_v, 0, nob_b0, sem0)
    _gather_start(act, idx_v, 0, act_b0, sem0)

    def pair(g, _):
        c0 = 2 * g
        c1 = 2 * g + 1
        _gather_start(obs, idx_v, c1, obs_b1, sem1)
        _gather_start(nobs, idx_v, c1, nob_b1, sem1)
        _gather_start(act, idx_v, c1, act_b1, sem1)
        lax.fori_loop(g * SG_PER_PAIR, (g + 1) * SG_PER_PAIR, sgather, 0)
        _gather_wait(obs, idx_v, c0, obs_b0, sem0)
        _gather_wait(nobs, idx_v, c0, nob_b0, sem0)
        _gather_wait(act, idx_v, c0, act_b0, sem0)
        o0 = row0 + c0 * CHUNK
        pltpu.sync_copy(obs_b0, obs_o.at[pl.ds(o0, CHUNK)])
        pltpu.sync_copy(nob_b0, nobs_o.at[pl.ds(o0, CHUNK)])
        pltpu.sync_copy(act_b0, act_o.at[pl.ds(o0, CHUNK)])

        @pl.when(g < NPAIR - 1)
        def _():
            _gather_start(obs, idx_v, c1 + 1, obs_b0, sem0)
            _gather_start(nobs, idx_v, c1 + 1, nob_b0, sem0)
            _gather_start(act, idx_v, c1 + 1, act_b0, sem0)

        _gather_wait(obs, idx_v, c1, obs_b1, sem1)
        _gather_wait(nobs, idx_v, c1, nob_b1, sem1)
        _gather_wait(act, idx_v, c1, act_b1, sem1)
        o1 = row0 + c1 * CHUNK
        pltpu.sync_copy(obs_b1, obs_o.at[pl.ds(o1, CHUNK)])
        pltpu.sync_copy(nob_b1, nobs_o.at[pl.ds(o1, CHUNK)])
        pltpu.sync_copy(act_b1, act_o.at[pl.ds(o1, CHUNK)])
        return 0

    lax.fori_loop(0, NPAIR, pair, 0)

    pltpu.sync_copy(rew_ov, rew_o.at[pl.ds(row0, RPW)])
    pltpu.sync_copy(don_ov, don_o.at[pl.ds(row0, RPW)])
    pltpu.sync_copy(trn_ov, trn_o.at[pl.ds(row0, RPW)])


def kernel(observations, actions, rewards, dones, truncations,
           next_observations, indices):
    obs = observations.reshape(N_ENV * BUF, N_OBS)
    nobs = next_observations.reshape(N_ENV * BUF, N_OBS)
    act = actions.reshape(N_ENV * BUF, N_ACT)
    rew = rewards.reshape(-1)
    don = dones.reshape(-1)
    trn = truncations.reshape(-1)
    idx = indices.reshape(-1)
    obs_s, act_s, rew_s, don_s, trn_s, nobs_s = _sample(
        obs, nobs, act, rew, don, trn, idx)
    eff = jnp.ones((B,), jnp.int32)
    return (obs_s, act_s, rew_s, don_s, trn_s, nobs_s, eff)
